# baseline (speedup 1.0000x reference)
import functools

import jax
import jax.numpy as jnp
from jax import lax
from jax.experimental import pallas as pl
from jax.experimental.pallas import tpu as pltpu
from jax.experimental.pallas import tpu_sc as plsc

VOCAB = 1000000
EMBED_DIM = 64
SKIP_DIM = 16
OUT_DIM = EMBED_DIM + SKIP_DIM

NC = 2
NS = 16
NW = NC * NS

IW = 128
NI = 4
CHUNK = NI * IW


def _body(x_hbm, st_hbm, item_hbm, skip_hbm, out_hbm,
          idx_v, stat_v, item_v, skip_v, sem_i, sem_s):
    n_rows = x_hbm.shape[0]
    rows_per_w = n_rows // NW
    n_chunks = rows_per_w // NI
    wid = lax.axis_index("s") * NC + lax.axis_index("c")
    r_base = wid * rows_per_w

    def chunk_body(g, carry):
        r0 = r_base + g * NI
        row0 = r0 * IW
        pltpu.sync_copy(x_hbm.at[pl.ds(r0, NI)], idx_v)
        pltpu.sync_copy(st_hbm.at[pl.ds(r0, NI)], stat_v)
        copies = []
        for j in range(NI):
            copies.append(pltpu.async_copy(
                item_hbm.at[idx_v.at[j]],
                item_v.at[pl.ds(j * IW, IW)], sem_i))
            copies.append(pltpu.async_copy(
                skip_hbm.at[stat_v.at[j]],
                skip_v.at[pl.ds(j * IW, IW)], sem_s))
        for c in copies:
            c.wait()
        pltpu.sync_copy(item_v, out_hbm.at[pl.ds(row0, CHUNK),
                                           pl.ds(0, EMBED_DIM)])
        pltpu.sync_copy(skip_v, out_hbm.at[pl.ds(row0, CHUNK),
                                           pl.ds(EMBED_DIM, SKIP_DIM)])
        return carry

    lax.fori_loop(0, n_chunks, chunk_body, 0)


def kernel(x, skip_status, item_table, skip_table):
    B, L = x.shape
    n = B * L
    x2d = x.reshape(n // IW, IW)
    s2d = skip_status.reshape(n // IW, IW)

    mesh = plsc.VectorSubcoreMesh(core_axis_name="c", subcore_axis_name="s")
    run = functools.partial(
        pl.kernel,
        out_type=jax.ShapeDtypeStruct((n, OUT_DIM), jnp.float32),
        mesh=mesh,
        scratch_types=[
            pltpu.VMEM((NI, IW), jnp.int32),
            pltpu.VMEM((NI, IW), jnp.int32),
            pltpu.VMEM((CHUNK, EMBED_DIM), jnp.float32),
            pltpu.VMEM((CHUNK, SKIP_DIM), jnp.float32),
            pltpu.SemaphoreType.DMA,
            pltpu.SemaphoreType.DMA,
        ],
        compiler_params=pltpu.CompilerParams(use_tc_tiling_on_sc=False),
    )(_body)
    out = run(x2d, s2d, item_table, skip_table)
    return out.reshape(B, L, OUT_DIM)


# speedup vs baseline: 3.7758x; 3.7758x over previous
import functools

import jax
import jax.numpy as jnp
from jax import lax
from jax.experimental import pallas as pl
from jax.experimental.pallas import tpu as pltpu
from jax.experimental.pallas import tpu_sc as plsc

VOCAB = 1000000
EMBED_DIM = 64
SKIP_DIM = 16
OUT_DIM = EMBED_DIM + SKIP_DIM

NC = 2
NS = 16
NW = NC * NS

IW = 128
NI = 4
CHUNK = NI * IW
NBUF = 2


def _body(x_hbm, st_hbm, item_hbm, skip_hbm, out_hbm,
          idx_bufs, st_bufs, item_bufs, skip_bufs, skip_vm,
          sem_idx, sem_g, sem_w):
    n_rows = x_hbm.shape[0]
    rows_per_w = n_rows // NW
    n_chunks = rows_per_w // NI
    n_super = n_chunks // NBUF
    wid = lax.axis_index("s") * NC + lax.axis_index("c")
    r_base = wid * rows_per_w

    pltpu.sync_copy(skip_hbm, skip_vm)
    t0 = skip_vm[0, :]
    t1 = skip_vm[1, :]

    def fire_idx(c, b):
        r0 = r_base + c * NI
        pltpu.async_copy(x_hbm.at[pl.ds(r0, NI)], idx_bufs[b], sem_idx[b])
        pltpu.async_copy(st_hbm.at[pl.ds(r0, NI)], st_bufs[b], sem_idx[b])

    for b in range(NBUF):
        fire_idx(b, b)

    def super_body(i, carry):
        for b in range(NBUF):
            c = i * NBUF + b
            r0 = r_base + c * NI
            pltpu.make_async_copy(x_hbm.at[pl.ds(r0, NI)],
                                  idx_bufs[b], sem_idx[b]).wait()
            pltpu.make_async_copy(st_hbm.at[pl.ds(r0, NI)],
                                  st_bufs[b], sem_idx[b]).wait()

            @pl.when(i >= 1)
            def _wait_prev():
                rp = (r_base + (c - NBUF) * NI) * IW
                pltpu.make_async_copy(
                    item_bufs[b],
                    out_hbm.at[pl.ds(rp, CHUNK), pl.ds(0, EMBED_DIM)],
                    sem_w[b]).wait()
                pltpu.make_async_copy(
                    skip_bufs[b],
                    out_hbm.at[pl.ds(rp, CHUNK), pl.ds(EMBED_DIM, SKIP_DIM)],
                    sem_w[b]).wait()

            gathers = []
            for j in range(NI):
                gathers.append(pltpu.async_copy(
                    item_hbm.at[idx_bufs[b].at[j]],
                    item_bufs[b].at[pl.ds(j * IW, IW)],
                    sem_g[b]))

            stat = st_bufs[b]

            def skip_row(r, carry2):
                s = plsc.load_gather(
                    stat, [jax.lax.div(r, IW) * jnp.ones((16,), jnp.int32),
                           jax.lax.rem(r, IW) * jnp.ones((16,), jnp.int32)])
                skip_bufs[b][r, :] = jnp.where(s != 0, t1, t0)
                return carry2

            lax.fori_loop(0, CHUNK, skip_row, 0, unroll=8)

            for g in gathers:
                g.wait()

            @pl.when(i < n_super - 1)
            def _prefetch():
                fire_idx(c + NBUF, b)

            pltpu.async_copy(
                item_bufs[b],
                out_hbm.at[pl.ds(r0 * IW, CHUNK), pl.ds(0, EMBED_DIM)],
                sem_w[b])
            pltpu.async_copy(
                skip_bufs[b],
                out_hbm.at[pl.ds(r0 * IW, CHUNK), pl.ds(EMBED_DIM, SKIP_DIM)],
                sem_w[b])
        return carry

    lax.fori_loop(0, n_super, super_body, 0)

    for b in range(NBUF):
        c = n_chunks - NBUF + b
        r0 = (r_base + c * NI) * IW
        pltpu.make_async_copy(
            item_bufs[b],
            out_hbm.at[pl.ds(r0, CHUNK), pl.ds(0, EMBED_DIM)],
            sem_w[b]).wait()
        pltpu.make_async_copy(
            skip_bufs[b],
            out_hbm.at[pl.ds(r0, CHUNK), pl.ds(EMBED_DIM, SKIP_DIM)],
            sem_w[b]).wait()


def _entry(x_hbm, st_hbm, item_hbm, skip_hbm, out_hbm, *scratch):
    idx_bufs = scratch[0:NBUF]
    st_bufs = scratch[NBUF:2 * NBUF]
    item_bufs = scratch[2 * NBUF:3 * NBUF]
    skip_bufs = scratch[3 * NBUF:4 * NBUF]
    skip_vm = scratch[4 * NBUF]
    sem_idx = scratch[4 * NBUF + 1:4 * NBUF + 1 + NBUF]
    sem_g = scratch[4 * NBUF + 1 + NBUF:4 * NBUF + 1 + 2 * NBUF]
    sem_w = scratch[4 * NBUF + 1 + 2 * NBUF:4 * NBUF + 1 + 3 * NBUF]
    _body(x_hbm, st_hbm, item_hbm, skip_hbm, out_hbm,
          idx_bufs, st_bufs, item_bufs, skip_bufs, skip_vm,
          sem_idx, sem_g, sem_w)


def kernel(x, skip_status, item_table, skip_table):
    B, L = x.shape
    n = B * L
    x2d = x.reshape(n // IW, IW)
    s2d = skip_status.reshape(n // IW, IW)

    mesh = plsc.VectorSubcoreMesh(core_axis_name="c", subcore_axis_name="s")
    scratch = (
        [pltpu.VMEM((NI, IW), jnp.int32) for _ in range(NBUF)] +
        [pltpu.VMEM((NI, IW), jnp.int32) for _ in range(NBUF)] +
        [pltpu.VMEM((CHUNK, EMBED_DIM), jnp.float32) for _ in range(NBUF)] +
        [pltpu.VMEM((CHUNK, SKIP_DIM), jnp.float32) for _ in range(NBUF)] +
        [pltpu.VMEM((2, SKIP_DIM), jnp.float32)] +
        [pltpu.SemaphoreType.DMA for _ in range(3 * NBUF)]
    )
    run = functools.partial(
        pl.kernel,
        out_type=jax.ShapeDtypeStruct((n, OUT_DIM), jnp.float32),
        mesh=mesh,
        scratch_types=scratch,
        compiler_params=pltpu.CompilerParams(use_tc_tiling_on_sc=False,
                                             needs_layout_passes=False),
    )(_entry)
    out = run(x2d, s2d, item_table, skip_table)
    return out.reshape(B, L, OUT_DIM)


# speedup vs baseline: 3.7799x; 1.0011x over previous
import functools

import jax
import jax.numpy as jnp
from jax import lax
from jax.experimental import pallas as pl
from jax.experimental.pallas import tpu as pltpu
from jax.experimental.pallas import tpu_sc as plsc

VOCAB = 1000000
EMBED_DIM = 64
SKIP_DIM = 16
OUT_DIM = EMBED_DIM + SKIP_DIM

NC = 2
NS = 16
NW = NC * NS

CB = 2
NBUF = 2
L_SPLITS = ((0, 104), (104, 96))


def _body(x_hbm, st_hbm, item_hbm, skip_hbm, out_hbm,
          idx_bufs, st_bufs, item_bufs, skip_bufs, skip_vm,
          sem_idx, sem_g, sem_w):
    B, L = x_hbm.shape
    b_per_w = B // NW
    n_chunks = b_per_w // CB
    n_super = n_chunks // NBUF
    wid = lax.axis_index("s") * NC + lax.axis_index("c")
    b_base = wid * b_per_w

    pltpu.sync_copy(skip_hbm, skip_vm)
    t0 = skip_vm[0, :]
    t1 = skip_vm[1, :]

    def fire_idx(c, b):
        b0 = b_base + c * CB
        pltpu.async_copy(x_hbm.at[pl.ds(b0, CB)], idx_bufs[b], sem_idx[b])
        pltpu.async_copy(st_hbm.at[pl.ds(b0, CB)], st_bufs[b], sem_idx[b])

    for b in range(NBUF):
        fire_idx(b, b)

    def super_body(i, carry):
        for b in range(NBUF):
            c = i * NBUF + b
            b0 = b_base + c * CB
            pltpu.make_async_copy(x_hbm.at[pl.ds(b0, CB)],
                                  idx_bufs[b], sem_idx[b]).wait()
            pltpu.make_async_copy(st_hbm.at[pl.ds(b0, CB)],
                                  st_bufs[b], sem_idx[b]).wait()

            @pl.when(i >= 1)
            def _wait_prev():
                bp = b_base + (c - NBUF) * CB
                pltpu.make_async_copy(
                    item_bufs[b],
                    out_hbm.at[pl.ds(bp, CB), :, pl.ds(0, EMBED_DIM)],
                    sem_w[b]).wait()
                pltpu.make_async_copy(
                    skip_bufs[b],
                    out_hbm.at[pl.ds(bp, CB), :, pl.ds(EMBED_DIM, SKIP_DIM)],
                    sem_w[b]).wait()

            gathers = []
            for bb in range(CB):
                for off, ln in L_SPLITS:
                    gathers.append(pltpu.async_copy(
                        item_hbm.at[idx_bufs[b].at[bb, pl.ds(off, ln)]],
                        item_bufs[b].at[bb, pl.ds(off, ln)],
                        sem_g[b]))

            stat = st_bufs[b]

            def skip_row(r, carry2):
                bb = jax.lax.div(r, L)
                l = jax.lax.rem(r, L)
                s = plsc.load_gather(
                    stat, [bb * jnp.ones((16,), jnp.int32),
                           l * jnp.ones((16,), jnp.int32)])
                skip_bufs[b][bb, l, :] = jnp.where(s != 0, t1, t0)
                return carry2

            lax.fori_loop(0, CB * L, skip_row, 0, unroll=8)

            for g in gathers:
                g.wait()

            @pl.when(i < n_super - 1)
            def _prefetch():
                fire_idx(c + NBUF, b)

            pltpu.async_copy(
                item_bufs[b],
                out_hbm.at[pl.ds(b0, CB), :, pl.ds(0, EMBED_DIM)],
                sem_w[b])
            pltpu.async_copy(
                skip_bufs[b],
                out_hbm.at[pl.ds(b0, CB), :, pl.ds(EMBED_DIM, SKIP_DIM)],
                sem_w[b])
        return carry

    lax.fori_loop(0, n_super, super_body, 0)

    for b in range(NBUF):
        c = n_chunks - NBUF + b
        b0 = b_base + c * CB
        pltpu.make_async_copy(
            item_bufs[b],
            out_hbm.at[pl.ds(b0, CB), :, pl.ds(0, EMBED_DIM)],
            sem_w[b]).wait()
        pltpu.make_async_copy(
            skip_bufs[b],
            out_hbm.at[pl.ds(b0, CB), :, pl.ds(EMBED_DIM, SKIP_DIM)],
            sem_w[b]).wait()


def _entry(x_hbm, st_hbm, item_hbm, skip_hbm, out_hbm, *scratch):
    idx_bufs = scratch[0:NBUF]
    st_bufs = scratch[NBUF:2 * NBUF]
    item_bufs = scratch[2 * NBUF:3 * NBUF]
    skip_bufs = scratch[3 * NBUF:4 * NBUF]
    skip_vm = scratch[4 * NBUF]
    sem_idx = scratch[4 * NBUF + 1:4 * NBUF + 1 + NBUF]
    sem_g = scratch[4 * NBUF + 1 + NBUF:4 * NBUF + 1 + 2 * NBUF]
    sem_w = scratch[4 * NBUF + 1 + 2 * NBUF:4 * NBUF + 1 + 3 * NBUF]
    _body(x_hbm, st_hbm, item_hbm, skip_hbm, out_hbm,
          idx_bufs, st_bufs, item_bufs, skip_bufs, skip_vm,
          sem_idx, sem_g, sem_w)


def kernel(x, skip_status, item_table, skip_table):
    B, L = x.shape
    mesh = plsc.VectorSubcoreMesh(core_axis_name="c", subcore_axis_name="s")
    scratch = (
        [pltpu.VMEM((CB, L), jnp.int32) for _ in range(NBUF)] +
        [pltpu.VMEM((CB, L), jnp.int32) for _ in range(NBUF)] +
        [pltpu.VMEM((CB, L, EMBED_DIM), jnp.float32) for _ in range(NBUF)] +
        [pltpu.VMEM((CB, L, SKIP_DIM), jnp.float32) for _ in range(NBUF)] +
        [pltpu.VMEM((2, SKIP_DIM), jnp.float32)] +
        [pltpu.SemaphoreType.DMA for _ in range(3 * NBUF)]
    )
    run = functools.partial(
        pl.kernel,
        out_type=jax.ShapeDtypeStruct((B, L, OUT_DIM), jnp.float32),
        mesh=mesh,
        scratch_types=scratch,
        compiler_params=pltpu.CompilerParams(use_tc_tiling_on_sc=False,
                                             needs_layout_passes=False),
    )(_entry)
    return run(x, skip_status, item_table, skip_table)
